# baseline (speedup 1.0000x reference)
import functools

import jax
import jax.numpy as jnp
from jax import lax
from jax.experimental import pallas as pl
from jax.experimental.pallas import tpu as pltpu
from jax.experimental.pallas import tpu_sc as plsc

E = 64
K = 2
D = 1024
F = 512
N = 4096
NFLAT = N * K
TBLK = 128
NT = NFLAT // TBLK
NSEG = NT + E
RBLK = 256

NW = 32
SC_SCAT_CHUNK = 64
SC_GATH_CHUNK = 64


def _rmeta_body(x_ref, wg_ref, sc_ref, dest_ref, exp_ref, hi_ref):
    xb = x_ref[...]
    logits = lax.dot_general(xb, wg_ref[...], (((1,), (1,)), ((), ())),
                             preferred_element_type=jnp.float32)
    m = jnp.max(logits, axis=1, keepdims=True)
    ex = jnp.exp(logits - m)
    p = ex / jnp.sum(ex, axis=1, keepdims=True)
    lane = lax.broadcasted_iota(jnp.int32, (N, E), 1)
    m1 = jnp.max(p, axis=1, keepdims=True)
    i1 = jnp.min(jnp.where(p == m1, lane, E), axis=1, keepdims=True)
    p2 = jnp.where(lane == i1, -1.0, p)
    m2 = jnp.max(p2, axis=1, keepdims=True)
    i2 = jnp.min(jnp.where(p2 == m2, lane, E), axis=1, keepdims=True)
    lane128 = lax.broadcasted_iota(jnp.int32, (N, 128), 1)
    sc_ref[...] = jnp.where(lane128 == 0, m1, jnp.where(lane128 == 1, m2, 0.0))

    oh1 = (lane == i1).astype(jnp.float32)
    oh2 = (lane == i2).astype(jnp.float32)
    t1 = jnp.transpose(oh1)
    t2 = jnp.transpose(oh2)
    A = jnp.concatenate([t1.reshape(E, 32, 128),
                         t2.reshape(E, 32, 128)], axis=1)

    r_i = lax.broadcasted_iota(jnp.int32, (128, 128), 0)
    c_i = lax.broadcasted_iota(jnp.int32, (128, 128), 1)
    Tinc = (r_i <= c_i).astype(jnp.float32)
    B = lax.dot_general(A, Tinc, (((2,), (0,)), ((), ())),
                        preferred_element_type=jnp.float32)
    R = B[:, :, 127]
    r64 = lax.broadcasted_iota(jnp.int32, (64, 64), 0)
    c64 = lax.broadcasted_iota(jnp.int32, (64, 64), 1)
    SL = (r64 < c64).astype(jnp.float32)
    S = lax.dot_general(R, SL, (((1,), (0,)), ((), ())),
                        preferred_element_type=jnp.float32)
    P = B + S[:, :, None]
    cnt_col = jnp.sum(R, axis=1, keepdims=True)
    SLT = (c64 < r64).astype(jnp.float32)

    tcnt_col = jnp.floor((cnt_col + float(TBLK - 1)) / float(TBLK))
    ptile_col = lax.dot_general(SLT, tcnt_col, (((1,), (0,)), ((), ())),
                                preferred_element_type=jnp.float32)
    pstart_col = ptile_col * float(TBLK)

    rank_incl = jnp.sum(A * P, axis=0)
    base = jnp.sum(A * pstart_col[:, :, None], axis=0)
    dest_ref[...] = (base + rank_incl - 1.0).astype(jnp.int32)

    g_row = lax.broadcasted_iota(jnp.int32, (1, 128), 1).astype(jnp.float32)
    expert_row = jnp.clip(
        jnp.sum((ptile_col <= g_row).astype(jnp.float32), axis=0,
                keepdims=True) - 1.0, 0.0, float(E - 1))
    e_col = lax.broadcasted_iota(jnp.int32, (E, 128), 0).astype(jnp.float32)
    ohg = (e_col == expert_row).astype(jnp.float32)
    ps_g = jnp.sum(ohg * ptile_col, axis=0, keepdims=True)
    cnt_g = jnp.sum(ohg * cnt_col, axis=0, keepdims=True)
    hi = jnp.clip(cnt_g - float(TBLK) * (g_row - ps_g), 0.0,
                  float(TBLK)).astype(jnp.int32)

    exp_ref[...] = jnp.broadcast_to(expert_row.astype(jnp.int32), (8, 128))
    hi_ref[...] = jnp.broadcast_to(hi, (8, 128))


def _rmeta(x, wg, *, interpret=False):
    return pl.pallas_call(
        _rmeta_body,
        out_shape=[
            jax.ShapeDtypeStruct((N, 128), jnp.float32),
            jax.ShapeDtypeStruct((64, 128), jnp.int32),
            jax.ShapeDtypeStruct((8, 128), jnp.int32),
            jax.ShapeDtypeStruct((8, 128), jnp.int32),
        ],
        interpret=interpret,
    )(x, wg)


def _sc_scatter_body(x_hbm, dest_hbm, perm_hbm, idx_v, rows_v, sem):
    c = lax.axis_index("c")
    s = lax.axis_index("s")
    wid = s * 2 + c
    base = wid * (NFLAT // NW)
    for k in range(NFLAT // NW // SC_SCAT_CHUNK):
        off = base + k * SC_SCAT_CHUNK
        pltpu.sync_copy(dest_hbm.at[pl.ds(off, SC_SCAT_CHUNK)], idx_v)
        pltpu.sync_copy(x_hbm.at[pl.ds(off % N, SC_SCAT_CHUNK)], rows_v)
        pltpu.async_copy(rows_v, perm_hbm.at[idx_v], sem).wait()


def _sc_scatter(x, dest):
    mesh = plsc.VectorSubcoreMesh(core_axis_name="c", subcore_axis_name="s")
    f = pl.kernel(
        _sc_scatter_body,
        out_type=jax.ShapeDtypeStruct((NSEG * TBLK, D), jnp.float32),
        mesh=mesh,
        scratch_types=[
            pltpu.VMEM((SC_SCAT_CHUNK,), jnp.int32),
            pltpu.VMEM((SC_SCAT_CHUNK, D), jnp.float32),
            pltpu.SemaphoreType.DMA,
        ],
    )
    return f(x, dest)


def _gemm_body(exp_r, hi_r, p_ref, w1_ref, w3_ref, w2_ref, y_ref,
               w1s, w3s, w2s):
    g = pl.program_id(0)

    @pl.when(hi_r[g] > 0)
    def _():
        prev = jnp.where(g == 0, -1, exp_r[jnp.maximum(g - 1, 0)])

        @pl.when(exp_r[g] != prev)
        def _():
            w1s[...] = w1_ref[0].astype(jnp.bfloat16)
            w3s[...] = w3_ref[0].astype(jnp.bfloat16)
            w2s[...] = w2_ref[0].astype(jnp.bfloat16)

        xb = p_ref[...].astype(jnp.bfloat16)
        h1 = lax.dot_general(xb, w1s[...], (((1,), (1,)), ((), ())),
                             preferred_element_type=jnp.float32)
        h3 = lax.dot_general(xb, w3s[...], (((1,), (1,)), ((), ())),
                             preferred_element_type=jnp.float32)
        h = (h1 * jax.nn.sigmoid(h1)) * h3
        o = lax.dot_general(h.astype(jnp.bfloat16), w2s[...],
                            (((1,), (1,)), ((), ())),
                            preferred_element_type=jnp.float32)
        rows = lax.broadcasted_iota(jnp.int32, (TBLK, D), 0)
        y_ref[...] = jnp.where(rows < hi_r[g], o, 0.0)


def _gemm(exp, hi, perm, w1b, w3b, w2b, *, interpret=False):
    grid_spec = pltpu.PrefetchScalarGridSpec(
        num_scalar_prefetch=2,
        grid=(NSEG,),
        in_specs=[
            pl.BlockSpec((TBLK, D),
                         lambda g, e, h: (jnp.where(h[g] > 0, g, 0), 0)),
            pl.BlockSpec((1, F, D), lambda g, e, h: (e[g], 0, 0)),
            pl.BlockSpec((1, F, D), lambda g, e, h: (e[g], 0, 0)),
            pl.BlockSpec((1, D, F), lambda g, e, h: (e[g], 0, 0)),
        ],
        out_specs=pl.BlockSpec((TBLK, D), lambda g, e, h: (g, 0)),
        scratch_shapes=[
            pltpu.VMEM((F, D), jnp.bfloat16),
            pltpu.VMEM((F, D), jnp.bfloat16),
            pltpu.VMEM((D, F), jnp.bfloat16),
        ],
    )
    return pl.pallas_call(
        _gemm_body,
        grid_spec=grid_spec,
        out_shape=jax.ShapeDtypeStruct((NSEG * TBLK, D), jnp.float32),
        interpret=interpret,
    )(exp, hi, perm, w1b, w3b, w2b)


def _sc_gather_body(y_hbm, ia_hbm, ib_hbm, ya_hbm, yb_hbm, idx_v, rows_v, sem):
    c = lax.axis_index("c")
    s = lax.axis_index("s")
    wid = s * 2 + c
    base = wid * (N // NW)
    for k in range(N // NW // SC_GATH_CHUNK):
        off = base + k * SC_GATH_CHUNK
        pltpu.sync_copy(ia_hbm.at[pl.ds(off, SC_GATH_CHUNK)], idx_v)
        pltpu.async_copy(y_hbm.at[idx_v], rows_v, sem).wait()
        pltpu.sync_copy(rows_v, ya_hbm.at[pl.ds(off, SC_GATH_CHUNK)])
        pltpu.sync_copy(ib_hbm.at[pl.ds(off, SC_GATH_CHUNK)], idx_v)
        pltpu.async_copy(y_hbm.at[idx_v], rows_v, sem).wait()
        pltpu.sync_copy(rows_v, yb_hbm.at[pl.ds(off, SC_GATH_CHUNK)])


def _sc_gather(y, ia, ib):
    mesh = plsc.VectorSubcoreMesh(core_axis_name="c", subcore_axis_name="s")
    f = pl.kernel(
        _sc_gather_body,
        out_type=[
            jax.ShapeDtypeStruct((N, D), jnp.float32),
            jax.ShapeDtypeStruct((N, D), jnp.float32),
        ],
        mesh=mesh,
        scratch_types=[
            pltpu.VMEM((SC_GATH_CHUNK,), jnp.int32),
            pltpu.VMEM((SC_GATH_CHUNK, D), jnp.float32),
            pltpu.SemaphoreType.DMA,
        ],
    )
    return f(y, ia, ib)


def _comb_body(ya_ref, yb_ref, s_ref, o_ref):
    s1 = s_ref[:, 0:1]
    s2 = s_ref[:, 1:2]
    o_ref[...] = ya_ref[...] * s1 + yb_ref[...] * s2


def _combine(ya, yb, scores, *, interpret=False):
    return pl.pallas_call(
        _comb_body,
        grid=(N // RBLK,),
        in_specs=[
            pl.BlockSpec((RBLK, D), lambda i: (i, 0)),
            pl.BlockSpec((RBLK, D), lambda i: (i, 0)),
            pl.BlockSpec((RBLK, 128), lambda i: (i, 0)),
        ],
        out_specs=pl.BlockSpec((RBLK, D), lambda i: (i, 0)),
        out_shape=jax.ShapeDtypeStruct((N, D), jnp.float32),
        interpret=interpret,
    )(ya, yb, scores)


def kernel(x, wg, w1, w2, w3):
    sc128, dest2d, exp8, hi8 = _rmeta(x, wg)
    dest = dest2d.reshape(NFLAT)
    exp, hi = exp8[0], hi8[0]

    perm = _sc_scatter(x, dest)

    y = _gemm(exp, hi, perm, w1, w3, w2)

    ya, yb = _sc_gather(y, dest[:N], dest[N:])
    return _combine(ya, yb, sc128)


# speedup vs baseline: 1.0327x; 1.0327x over previous
import functools

import jax
import jax.numpy as jnp
from jax import lax
from jax.experimental import pallas as pl
from jax.experimental.pallas import tpu as pltpu
from jax.experimental.pallas import tpu_sc as plsc

E = 64
K = 2
D = 1024
F = 512
N = 4096
NFLAT = N * K
TBLK = 128
NT = NFLAT // TBLK
NSEG = NT + E
RBLK = 256

NW = 32
SC_SCAT_CHUNK = 64
SC_GATH_CHUNK = 64


def _rmeta_body(x_ref, wg_ref, sc_ref, dest_ref, exp_ref, hi_ref):
    xb = x_ref[...]
    logits = lax.dot_general(xb, wg_ref[...], (((1,), (1,)), ((), ())),
                             preferred_element_type=jnp.float32)
    m = jnp.max(logits, axis=1, keepdims=True)
    ex = jnp.exp(logits - m)
    p = ex / jnp.sum(ex, axis=1, keepdims=True)
    lane = lax.broadcasted_iota(jnp.int32, (N, E), 1)
    m1 = jnp.max(p, axis=1, keepdims=True)
    i1 = jnp.min(jnp.where(p == m1, lane, E), axis=1, keepdims=True)
    p2 = jnp.where(lane == i1, -1.0, p)
    m2 = jnp.max(p2, axis=1, keepdims=True)
    i2 = jnp.min(jnp.where(p2 == m2, lane, E), axis=1, keepdims=True)
    lane128 = lax.broadcasted_iota(jnp.int32, (N, 128), 1)
    sc_ref[...] = jnp.where(lane128 == 0, m1, jnp.where(lane128 == 1, m2, 0.0))

    oh1 = (lane == i1).astype(jnp.float32)
    oh2 = (lane == i2).astype(jnp.float32)
    t1 = jnp.transpose(oh1)
    t2 = jnp.transpose(oh2)
    A = jnp.concatenate([t1.reshape(E, 32, 128),
                         t2.reshape(E, 32, 128)], axis=1)

    r_i = lax.broadcasted_iota(jnp.int32, (128, 128), 0)
    c_i = lax.broadcasted_iota(jnp.int32, (128, 128), 1)
    Tinc = (r_i <= c_i).astype(jnp.float32)
    B = lax.dot_general(A, Tinc, (((2,), (0,)), ((), ())),
                        preferred_element_type=jnp.float32)
    R = B[:, :, 127]
    r64 = lax.broadcasted_iota(jnp.int32, (64, 64), 0)
    c64 = lax.broadcasted_iota(jnp.int32, (64, 64), 1)
    SL = (r64 < c64).astype(jnp.float32)
    S = lax.dot_general(R, SL, (((1,), (0,)), ((), ())),
                        preferred_element_type=jnp.float32)
    P = B + S[:, :, None]
    cnt_col = jnp.sum(R, axis=1, keepdims=True)
    SLT = (c64 < r64).astype(jnp.float32)

    tcnt_col = jnp.floor((cnt_col + float(TBLK - 1)) / float(TBLK))
    ptile_col = lax.dot_general(SLT, tcnt_col, (((1,), (0,)), ((), ())),
                                preferred_element_type=jnp.float32)
    pstart_col = ptile_col * float(TBLK)

    rank_incl = jnp.sum(A * P, axis=0)
    base = jnp.sum(A * pstart_col[:, :, None], axis=0)
    dest_ref[...] = (base + rank_incl - 1.0).astype(jnp.int32)

    g_row = lax.broadcasted_iota(jnp.int32, (1, 128), 1).astype(jnp.float32)
    expert_row = jnp.clip(
        jnp.sum((ptile_col <= g_row).astype(jnp.float32), axis=0,
                keepdims=True) - 1.0, 0.0, float(E - 1))
    e_col = lax.broadcasted_iota(jnp.int32, (E, 128), 0).astype(jnp.float32)
    ohg = (e_col == expert_row).astype(jnp.float32)
    ps_g = jnp.sum(ohg * ptile_col, axis=0, keepdims=True)
    cnt_g = jnp.sum(ohg * cnt_col, axis=0, keepdims=True)
    hi = jnp.clip(cnt_g - float(TBLK) * (g_row - ps_g), 0.0,
                  float(TBLK)).astype(jnp.int32)

    exp_ref[...] = jnp.broadcast_to(expert_row.astype(jnp.int32), (8, 128))
    hi_ref[...] = jnp.broadcast_to(hi, (8, 128))


def _rmeta(x, wg, *, interpret=False):
    return pl.pallas_call(
        _rmeta_body,
        out_shape=[
            jax.ShapeDtypeStruct((N, 128), jnp.float32),
            jax.ShapeDtypeStruct((64, 128), jnp.int32),
            jax.ShapeDtypeStruct((8, 128), jnp.int32),
            jax.ShapeDtypeStruct((8, 128), jnp.int32),
        ],
        interpret=interpret,
    )(x, wg)


def _sc_scatter_body(x_hbm, dest_hbm, perm_hbm, idx_v, rows_v, sem):
    c = lax.axis_index("c")
    s = lax.axis_index("s")
    wid = s * 2 + c
    base = wid * (NFLAT // NW)
    for k in range(NFLAT // NW // SC_SCAT_CHUNK):
        off = base + k * SC_SCAT_CHUNK
        pltpu.sync_copy(dest_hbm.at[pl.ds(off, SC_SCAT_CHUNK)], idx_v)
        pltpu.sync_copy(x_hbm.at[pl.ds(off % N, SC_SCAT_CHUNK)], rows_v)
        pltpu.async_copy(rows_v, perm_hbm.at[idx_v], sem).wait()


def _sc_scatter(x, dest):
    mesh = plsc.VectorSubcoreMesh(core_axis_name="c", subcore_axis_name="s")
    f = pl.kernel(
        _sc_scatter_body,
        out_type=jax.ShapeDtypeStruct((NSEG * TBLK, D), jnp.float32),
        mesh=mesh,
        scratch_types=[
            pltpu.VMEM((SC_SCAT_CHUNK,), jnp.int32),
            pltpu.VMEM((SC_SCAT_CHUNK, D), jnp.float32),
            pltpu.SemaphoreType.DMA,
        ],
    )
    return f(x, dest)


def _gemm_body(exp_r, hi_r, p_ref, w1_ref, w3_ref, w2_ref, y_ref):
    g = pl.program_id(0)

    @pl.when(hi_r[g] > 0)
    def _():
        xb = p_ref[...].astype(jnp.bfloat16)
        w1b = w1_ref[0].astype(jnp.bfloat16)
        w3b = w3_ref[0].astype(jnp.bfloat16)
        w2b = w2_ref[0].astype(jnp.bfloat16)
        h1 = lax.dot_general(xb, w1b, (((1,), (1,)), ((), ())),
                             preferred_element_type=jnp.float32)
        h3 = lax.dot_general(xb, w3b, (((1,), (1,)), ((), ())),
                             preferred_element_type=jnp.float32)
        h = (h1 * jax.nn.sigmoid(h1)) * h3
        o = lax.dot_general(h.astype(jnp.bfloat16), w2b,
                            (((1,), (1,)), ((), ())),
                            preferred_element_type=jnp.float32)
        rows = lax.broadcasted_iota(jnp.int32, (TBLK, D), 0)
        y_ref[...] = jnp.where(rows < hi_r[g], o, 0.0)


def _gemm(exp, hi, perm, w1b, w3b, w2b, *, interpret=False):
    grid_spec = pltpu.PrefetchScalarGridSpec(
        num_scalar_prefetch=2,
        grid=(NSEG,),
        in_specs=[
            pl.BlockSpec((TBLK, D),
                         lambda g, e, h: (jnp.where(h[g] > 0, g, 0), 0)),
            pl.BlockSpec((1, F, D), lambda g, e, h: (e[g], 0, 0)),
            pl.BlockSpec((1, F, D), lambda g, e, h: (e[g], 0, 0)),
            pl.BlockSpec((1, D, F), lambda g, e, h: (e[g], 0, 0)),
        ],
        out_specs=pl.BlockSpec((TBLK, D), lambda g, e, h: (g, 0)),
    )
    return pl.pallas_call(
        _gemm_body,
        grid_spec=grid_spec,
        out_shape=jax.ShapeDtypeStruct((NSEG * TBLK, D), jnp.float32),
        interpret=interpret,
    )(exp, hi, perm, w1b, w3b, w2b)


def _sc_gather_body(y_hbm, ia_hbm, ib_hbm, ya_hbm, yb_hbm, idx_v, rows_v, sem):
    c = lax.axis_index("c")
    s = lax.axis_index("s")
    wid = s * 2 + c
    base = wid * (N // NW)
    for k in range(N // NW // SC_GATH_CHUNK):
        off = base + k * SC_GATH_CHUNK
        pltpu.sync_copy(ia_hbm.at[pl.ds(off, SC_GATH_CHUNK)], idx_v)
        pltpu.async_copy(y_hbm.at[idx_v], rows_v, sem).wait()
        pltpu.sync_copy(rows_v, ya_hbm.at[pl.ds(off, SC_GATH_CHUNK)])
        pltpu.sync_copy(ib_hbm.at[pl.ds(off, SC_GATH_CHUNK)], idx_v)
        pltpu.async_copy(y_hbm.at[idx_v], rows_v, sem).wait()
        pltpu.sync_copy(rows_v, yb_hbm.at[pl.ds(off, SC_GATH_CHUNK)])


def _sc_gather(y, ia, ib):
    mesh = plsc.VectorSubcoreMesh(core_axis_name="c", subcore_axis_name="s")
    f = pl.kernel(
        _sc_gather_body,
        out_type=[
            jax.ShapeDtypeStruct((N, D), jnp.float32),
            jax.ShapeDtypeStruct((N, D), jnp.float32),
        ],
        mesh=mesh,
        scratch_types=[
            pltpu.VMEM((SC_GATH_CHUNK,), jnp.int32),
            pltpu.VMEM((SC_GATH_CHUNK, D), jnp.float32),
            pltpu.SemaphoreType.DMA,
        ],
    )
    return f(y, ia, ib)


def _comb_body(ya_ref, yb_ref, s_ref, o_ref):
    s1 = s_ref[:, 0:1]
    s2 = s_ref[:, 1:2]
    o_ref[...] = ya_ref[...] * s1 + yb_ref[...] * s2


def _combine(ya, yb, scores, *, interpret=False):
    return pl.pallas_call(
        _comb_body,
        grid=(N // RBLK,),
        in_specs=[
            pl.BlockSpec((RBLK, D), lambda i: (i, 0)),
            pl.BlockSpec((RBLK, D), lambda i: (i, 0)),
            pl.BlockSpec((RBLK, 128), lambda i: (i, 0)),
        ],
        out_specs=pl.BlockSpec((RBLK, D), lambda i: (i, 0)),
        out_shape=jax.ShapeDtypeStruct((N, D), jnp.float32),
        interpret=interpret,
    )(ya, yb, scores)


def kernel(x, wg, w1, w2, w3):
    sc128, dest2d, exp8, hi8 = _rmeta(x, wg)
    dest = dest2d.reshape(NFLAT)
    exp, hi = exp8[0], hi8[0]

    perm = _sc_scatter(x, dest)

    y = _gemm(exp, hi, perm, w1, w3, w2)

    ya, yb = _sc_gather(y, dest[:N], dest[N:])
    return _combine(ya, yb, sc128)
